# baseline (speedup 1.0000x reference)
import jax
import jax.numpy as jnp
from jax import lax
from jax.experimental import pallas as pl
from jax.experimental.pallas import tpu as pltpu
from jax.experimental.pallas import tpu_sc as plsc

NC = 2
NS = 16
NW = NC * NS

D = 33
NROWS = 1000000
NB = 16384
K = 10
RPW = NB // NW


def _gather_body(idx_hbm, table_hbm, out_hbm, idx_v, plane_v, gsem):
    wid = lax.axis_index("s") * NC + lax.axis_index("c")
    r0 = wid * RPW
    pltpu.sync_copy(idx_hbm.at[:, pl.ds(r0, RPW)], idx_v)

    for k in range(K):
        def fire(c, carry):
            pltpu.async_copy(
                table_hbm.at[c].at[idx_v.at[k]], plane_v.at[c], gsem
            )
            return carry

        def drain(c, carry):
            pltpu.make_async_copy(
                table_hbm.at[0].at[idx_v.at[k]], plane_v.at[c], gsem
            ).wait()
            return carry

        lax.fori_loop(0, D, fire, 0)
        lax.fori_loop(0, D, drain, 0)
        pltpu.sync_copy(plane_v, out_hbm.at[k].at[:, pl.ds(r0, RPW)])


def kernel(idx, embedding):
    emb_t = embedding.T
    idx_t = idx.T.astype(jnp.int32)
    idx_p16 = jnp.pad(idx_t, ((0, 16 - K), (0, 0)))
    mesh = plsc.VectorSubcoreMesh(
        core_axis_name="c", subcore_axis_name="s", num_cores=NC, num_subcores=NS
    )
    out_p = pl.kernel(
        _gather_body,
        out_type=jax.ShapeDtypeStruct((K, 40, NB), jnp.float32),
        mesh=mesh,
        scratch_types=[
            pltpu.VMEM((16, RPW), jnp.int32),
            pltpu.VMEM((40, RPW), jnp.float32),
            pltpu.SemaphoreType.DMA,
        ],
        compiler_params=pltpu.CompilerParams(use_tc_tiling_on_sc=False),
    )(idx_p16, emb_t)
    return out_p[:, :D, :].transpose(2, 0, 1)


# speedup vs baseline: 2.6561x; 2.6561x over previous
import jax
import jax.numpy as jnp
from jax import lax
from jax.experimental import pallas as pl
from jax.experimental.pallas import tpu as pltpu
from jax.experimental.pallas import tpu_sc as plsc

NC = 2
NS = 16
NW = NC * NS

D = 33
NROWS = 1000000
CHUNK = 128
B_TOTAL = 16384 * 10
PER_W = B_TOTAL // NW
NCHUNK = PER_W // CHUNK
NBUF = 5
L1 = 2
NROUND = NCHUNK // NBUF


def _gather_body(idx_hbm, tbl_hbm, out_hbm, idx_v, rows_v, gsem, osem):
    wid = lax.axis_index("s") * NC + lax.axis_index("c")
    pltpu.sync_copy(idx_hbm.at[wid], idx_v)
    base = wid * PER_W

    def gather(j, s):
        pltpu.async_copy(tbl_hbm.at[idx_v.at[j]], rows_v.at[s], gsem.at[s])

    def out_copy(j, s):
        pltpu.async_copy(
            rows_v.at[s], out_hbm.at[pl.ds(base + j * CHUNK, CHUNK)], osem.at[s]
        )

    def wait_gather(s):
        pltpu.make_async_copy(
            tbl_hbm.at[idx_v.at[0]], rows_v.at[s], gsem.at[s]
        ).wait()

    def wait_out(s):
        pltpu.make_async_copy(
            rows_v.at[s], out_hbm.at[pl.ds(base, CHUNK)], osem.at[s]
        ).wait()

    for b in range(L1):
        gather(b, b)

    def round_fn(r, carry):
        for b in range(NBUF):
            j = r * NBUF + b
            s_new = (b + L1) % NBUF
            j_new = j + L1
            if b < NBUF - L1:
                @pl.when(r >= 1)
                def _():
                    wait_out(s_new)
                    gather(j_new, s_new)

                @pl.when(r == 0)
                def _():
                    gather(j_new, s_new)
            else:
                @pl.when(r < NROUND - 1)
                def _():
                    wait_out(s_new)
                    gather(j_new, s_new)

            wait_gather(b)
            out_copy(j, b)
        return carry

    lax.fori_loop(0, NROUND, round_fn, 0)

    for b in range(NBUF):
        wait_out(b)


def kernel(idx, embedding):
    idx3 = idx.reshape(NW, NCHUNK, CHUNK)
    dep = jnp.pad(embedding, ((0, 0), (0, 128 - D)))
    mesh = plsc.VectorSubcoreMesh(
        core_axis_name="c", subcore_axis_name="s", num_cores=NC, num_subcores=NS
    )
    out128 = pl.kernel(
        _gather_body,
        out_type=jax.ShapeDtypeStruct((B_TOTAL, 128), jnp.float32),
        mesh=mesh,
        scratch_types=[
            pltpu.VMEM((NCHUNK, CHUNK), jnp.int32),
            pltpu.VMEM((NBUF, CHUNK, 128), jnp.float32),
            pltpu.SemaphoreType.DMA((NBUF,)),
            pltpu.SemaphoreType.DMA((NBUF,)),
        ],
    )(idx3, dep)
    return out128[:, :D].reshape(idx.shape[0], idx.shape[1], D)
